# baseline (speedup 1.0000x reference)
import functools

import jax
import jax.numpy as jnp
from jax import lax
from jax.experimental import pallas as pl
from jax.experimental.pallas import tpu as pltpu
from jax.experimental.pallas import tpu_sc as plsc

VOCAB = 1000000
EMBED_DIM = 64
BATCH = 4096
HIST = 50

N = BATCH * HIST
NC = 2
NS = 16
NW = NC * NS
LANES = 16

TCOLS = VOCAB // 128
TPW = TCOLS // NW
XCOLS = TCOLS - TPW * NW
TAIL_ROWS = VOCAB - TCOLS * 128
TAIL_WORDS = TAIL_ROWS * EMBED_DIM

PER_W = N // NW
CHUNK = 800
NCHUNK = PER_W // CHUNK

_MESH = plsc.VectorSubcoreMesh(core_axis_name="c", subcore_axis_name="s")


def _transpose_body(tt_hbm, tail_hbm, out_hbm, v0, v1, v2, v3, o0, o1, tl,
                    s0, s1, s2, s3):
    wid = lax.axis_index("s") * NC + lax.axis_index("c")
    vbufs, obufs, sems = (v0, v1, v2, v3), (o0, o1), (s0, s1, s2, s3)
    iota = lax.iota(jnp.int32, LANES)
    cidx = [iota + 16 * q for q in range(4)]

    def col_of(t):
        return wid + NW * t

    def start_load(b, k):
        pltpu.async_copy(tt_hbm.at[:, pl.ds(128 * k, 128)], vbufs[b], sems[b])

    def permute_write(b, k):
        v, o = vbufs[b], obufs[b & 1]

        def row(i, _):
            le = jnp.full((LANES,), 2 * i, jnp.int32)
            lo = jnp.full((LANES,), 2 * i + 1, jnp.int32)
            off = i * 128
            for q in range(4):
                o[pl.ds(off + 16 * q, 16)] = plsc.load_gather(v, [cidx[q], le])
                o[pl.ds(off + 64 + 16 * q, 16)] = plsc.load_gather(
                    v, [cidx[q], lo])
            return 0

        lax.fori_loop(0, 64, row, 0)
        pltpu.sync_copy(o, out_hbm.at[pl.ds(8192 * k, 8192)])

    for b in range(4):
        start_load(b, col_of(b))

    def steady_full(m, _):
        for b in range(4):
            t = 4 * m + b
            k = col_of(t)
            pltpu.make_async_copy(tt_hbm.at[:, pl.ds(128 * k, 128)],
                                  vbufs[b], sems[b]).wait()
            permute_write(b, k)
            pltpu.async_copy(tt_hbm.at[:, pl.ds(128 * col_of(t + 4), 128)],
                             vbufs[b], sems[b])
        return 0

    lax.fori_loop(0, TPW // 4 - 1, steady_full, 0)
    for t in range(TPW - 4, TPW):
        b = t & 3
        k = col_of(t)
        pltpu.make_async_copy(tt_hbm.at[:, pl.ds(128 * k, 128)],
                              vbufs[b], sems[b]).wait()
        permute_write(b, k)

    @pl.when(wid < XCOLS)
    def _():
        k = TPW * NW + wid
        pltpu.sync_copy(tt_hbm.at[:, pl.ds(128 * k, 128)], v0)
        permute_write(0, k)

    @pl.when(wid == NW - 1)
    def _():
        pltpu.sync_copy(tail_hbm, tl)
        pltpu.sync_copy(tl, out_hbm.at[pl.ds(8192 * TCOLS, TAIL_WORDS)])


_sc_transpose = functools.partial(
    pl.kernel,
    out_type=jax.ShapeDtypeStruct((VOCAB * EMBED_DIM,), jnp.float32),
    mesh=_MESH,
    scratch_types=[
        pltpu.VMEM((EMBED_DIM, 128), jnp.float32),
        pltpu.VMEM((EMBED_DIM, 128), jnp.float32),
        pltpu.VMEM((EMBED_DIM, 128), jnp.float32),
        pltpu.VMEM((EMBED_DIM, 128), jnp.float32),
        pltpu.VMEM((8192,), jnp.float32),
        pltpu.VMEM((8192,), jnp.float32),
        pltpu.VMEM((TAIL_WORDS,), jnp.float32),
        pltpu.SemaphoreType.DMA,
        pltpu.SemaphoreType.DMA,
        pltpu.SemaphoreType.DMA,
        pltpu.SemaphoreType.DMA,
    ],
    compiler_params=pltpu.CompilerParams(use_tc_tiling_on_sc=True,
                                         needs_layout_passes=False),
)(_transpose_body)


def _gather_body(x1_hbm, x2_hbm, table_hbm, out1_hbm, out2_hbm,
                 idx0, idx1, rows0, rows1, sem0, sem1):
    wid = lax.axis_index("s") * NC + lax.axis_index("c")
    base = wid * PER_W

    idx_bufs = (idx0, idx1)
    row_bufs = (rows0, rows1)
    sems = (sem0, sem1)

    sched = [(x1_hbm, out1_hbm, c) for c in range(NCHUNK)]
    sched += [(x2_hbm, out2_hbm, c) for c in range(NCHUNK)]

    handles = [None, None]
    for k, (src, dst, c) in enumerate(sched):
        b = k & 1
        off = base + c * CHUNK
        pltpu.sync_copy(src.at[pl.ds(off, CHUNK)], idx_bufs[b])
        handles[b] = pltpu.async_copy(table_hbm.at[idx_bufs[b]],
                                      row_bufs[b], sems[b])
        if k > 0:
            pb = 1 - b
            _, pdst, pc = sched[k - 1]
            handles[pb].wait()
            pltpu.sync_copy(row_bufs[pb],
                            pdst.at[pl.ds(base + pc * CHUNK, CHUNK)])
    lb = (len(sched) - 1) & 1
    _, ldst, lc = sched[-1]
    handles[lb].wait()
    pltpu.sync_copy(row_bufs[lb], ldst.at[pl.ds(base + lc * CHUNK, CHUNK)])


_sc_gather = functools.partial(
    pl.kernel,
    out_type=(jax.ShapeDtypeStruct((N, EMBED_DIM), jnp.float32),
              jax.ShapeDtypeStruct((N, EMBED_DIM), jnp.float32)),
    mesh=_MESH,
    scratch_types=[
        pltpu.VMEM((CHUNK,), jnp.int32),
        pltpu.VMEM((CHUNK,), jnp.int32),
        pltpu.VMEM((CHUNK, EMBED_DIM), jnp.float32),
        pltpu.VMEM((CHUNK, EMBED_DIM), jnp.float32),
        pltpu.SemaphoreType.DMA,
        pltpu.SemaphoreType.DMA,
    ],
    compiler_params=pltpu.CompilerParams(use_tc_tiling_on_sc=False),
)(_gather_body)


def kernel(x1, x2, table):
    tt = table.T
    tail = table[TCOLS * 128:, :].reshape(-1)
    flat = _sc_transpose(tt, tail)
    tl = flat.reshape(VOCAB, EMBED_DIM)
    f1 = x1.reshape(-1).astype(jnp.int32)
    f2 = x2.reshape(-1).astype(jnp.int32)
    o1, o2 = _sc_gather(f1, f2, tl)
    return (o1.reshape(BATCH, HIST, EMBED_DIM),
            o2.reshape(BATCH, HIST, EMBED_DIM))


# speedup vs baseline: 1.5939x; 1.5939x over previous
import functools

import jax
import jax.numpy as jnp
from jax import lax
from jax.experimental import pallas as pl
from jax.experimental.pallas import tpu as pltpu
from jax.experimental.pallas import tpu_sc as plsc

VOCAB = 1000000
EMBED_DIM = 64
BATCH = 4096
HIST = 50

N = BATCH * HIST
NC = 2
NS = 16
NW = NC * NS
PER_W = N // NW
CHUNK = 800
NCHUNK = PER_W // CHUNK

NBLK = VOCAB // 128
BPW = NBLK // NW
XBLK = NBLK - BPW * NW
TAIL_ROWS = VOCAB - NBLK * 128

_MESH = plsc.VectorSubcoreMesh(core_axis_name="c", subcore_axis_name="s")


def _repack(va, vb, nrows):
    def row(i, _):
        for q in range(4):
            vb[i >> 1, pl.ds(((i & 1) * 4 + q) * 16, 16)] = \
                va[i, pl.ds(16 * q, 16)]
        return 0

    lax.fori_loop(0, nrows, row, 0)


def _detile_body(tab_hbm, out_hbm, va0, va1, vb0, vb1, s0, s1):
    wid = lax.axis_index("s") * NC + lax.axis_index("c")
    vas, vbs, sems = (va0, va1), (vb0, vb1), (s0, s1)

    def blk_of(t):
        return wid + NW * t

    def start(b, g):
        pltpu.async_copy(tab_hbm.at[pl.ds(128 * g, 128), :], vas[b], sems[b])

    def finish(b, g):
        pltpu.make_async_copy(tab_hbm.at[pl.ds(128 * g, 128), :], vas[b],
                              sems[b]).wait()
        _repack(vas[b], vbs[b], 128)

    start(0, blk_of(0))
    start(1, blk_of(1))

    def steady(m, _):
        for b in range(2):
            t = 2 * m + b
            g = blk_of(t)
            finish(b, g)
            start(b, blk_of(t + 2))
            pltpu.sync_copy(vbs[b], out_hbm.at[pl.ds(64 * g, 64), :])
        return 0

    lax.fori_loop(0, BPW // 2 - 1, steady, 0)
    for t in (BPW - 2, BPW - 1):
        b = t & 1
        g = blk_of(t)
        finish(b, g)
        pltpu.sync_copy(vbs[b], out_hbm.at[pl.ds(64 * g, 64), :])

    @pl.when(wid < XBLK)
    def _():
        g = BPW * NW + wid
        pltpu.sync_copy(tab_hbm.at[pl.ds(128 * g, 128), :], va0)
        _repack(va0, vb0, 128)
        pltpu.sync_copy(vb0, out_hbm.at[pl.ds(64 * g, 64), :])

    @pl.when(wid == XBLK)
    def _():
        base = NBLK * 128
        pltpu.sync_copy(tab_hbm.at[pl.ds(base, TAIL_ROWS), :],
                        va1.at[pl.ds(0, TAIL_ROWS), :])
        _repack(va1, vb1, TAIL_ROWS)
        pltpu.sync_copy(vb1.at[pl.ds(0, TAIL_ROWS // 2), :],
                        out_hbm.at[pl.ds(base // 2, TAIL_ROWS // 2), :])


_sc_detile = functools.partial(
    pl.kernel,
    out_type=jax.ShapeDtypeStruct((VOCAB // 2, 128), jnp.float32),
    mesh=_MESH,
    scratch_types=[
        pltpu.VMEM((128, EMBED_DIM), jnp.float32),
        pltpu.VMEM((128, EMBED_DIM), jnp.float32),
        pltpu.VMEM((64, 128), jnp.float32),
        pltpu.VMEM((64, 128), jnp.float32),
        pltpu.SemaphoreType.DMA,
        pltpu.SemaphoreType.DMA,
    ],
    compiler_params=pltpu.CompilerParams(use_tc_tiling_on_sc=True,
                                         needs_layout_passes=False),
)(_detile_body)


def _body(x1_hbm, x2_hbm, table_hbm, out1_hbm, out2_hbm,
          idx0, idx1, rows0, rows1, sem0, sem1):
    wid = lax.axis_index("s") * NC + lax.axis_index("c")
    base = wid * PER_W

    idx_bufs = (idx0, idx1)
    row_bufs = (rows0, rows1)
    sems = (sem0, sem1)

    sched = [(x1_hbm, out1_hbm, c) for c in range(NCHUNK)]
    sched += [(x2_hbm, out2_hbm, c) for c in range(NCHUNK)]

    handles = [None, None]
    for k, (src, dst, c) in enumerate(sched):
        b = k & 1
        off = base + c * CHUNK
        pltpu.sync_copy(src.at[pl.ds(off, CHUNK)], idx_bufs[b])
        handles[b] = pltpu.async_copy(table_hbm.at[idx_bufs[b]],
                                      row_bufs[b], sems[b])
        if k > 0:
            pb = 1 - b
            _, pdst, pc = sched[k - 1]
            handles[pb].wait()
            pltpu.sync_copy(row_bufs[pb],
                            pdst.at[pl.ds(base + pc * CHUNK, CHUNK)])
    lb = (len(sched) - 1) & 1
    _, ldst, lc = sched[-1]
    handles[lb].wait()
    pltpu.sync_copy(row_bufs[lb], ldst.at[pl.ds(base + lc * CHUNK, CHUNK)])


_sc_kernel = functools.partial(
    pl.kernel,
    out_type=(jax.ShapeDtypeStruct((N, EMBED_DIM), jnp.float32),
              jax.ShapeDtypeStruct((N, EMBED_DIM), jnp.float32)),
    mesh=_MESH,
    scratch_types=[
        pltpu.VMEM((CHUNK,), jnp.int32),
        pltpu.VMEM((CHUNK,), jnp.int32),
        pltpu.VMEM((CHUNK, EMBED_DIM), jnp.float32),
        pltpu.VMEM((CHUNK, EMBED_DIM), jnp.float32),
        pltpu.SemaphoreType.DMA,
        pltpu.SemaphoreType.DMA,
    ],
    compiler_params=pltpu.CompilerParams(use_tc_tiling_on_sc=False),
)(_body)


def kernel(x1, x2, table):
    half = _sc_detile(table)
    tl = half.reshape(VOCAB, EMBED_DIM)
    f1 = x1.reshape(-1).astype(jnp.int32)
    f2 = x2.reshape(-1).astype(jnp.int32)
    o1, o2 = _sc_kernel(f1, f2, tl)
    return (o1.reshape(BATCH, HIST, EMBED_DIM),
            o2.reshape(BATCH, HIST, EMBED_DIM))


# speedup vs baseline: 2.0679x; 1.2974x over previous
import functools

import jax
import jax.numpy as jnp
from jax import lax
from jax.experimental import pallas as pl
from jax.experimental.pallas import tpu as pltpu
from jax.experimental.pallas import tpu_sc as plsc

VOCAB = 1000000
EMBED_DIM = 64
BATCH = 4096
HIST = 50

N = BATCH * HIST
NC = 2
NS = 16
NW = NC * NS
PER_W = N // NW
CHUNK = 800
NCHUNK = PER_W // CHUNK

_MESH = plsc.VectorSubcoreMesh(core_axis_name="c", subcore_axis_name="s")


def _body(x1_hbm, x2_hbm, table_hbm, out1_hbm, out2_hbm,
          idx0, idx1, rows0, rows1, sem0, sem1):
    wid = lax.axis_index("s") * NC + lax.axis_index("c")
    base = wid * PER_W

    idx_bufs = (idx0, idx1)
    row_bufs = (rows0, rows1)
    sems = (sem0, sem1)

    sched = [(x1_hbm, out1_hbm, c) for c in range(NCHUNK)]
    sched += [(x2_hbm, out2_hbm, c) for c in range(NCHUNK)]

    handles = [None, None]
    for k, (src, dst, c) in enumerate(sched):
        b = k & 1
        off = base + c * CHUNK
        pltpu.sync_copy(src.at[pl.ds(off, CHUNK)], idx_bufs[b])
        handles[b] = pltpu.async_copy(table_hbm.at[idx_bufs[b]],
                                      row_bufs[b], sems[b])
        if k > 0:
            pb = 1 - b
            _, pdst, pc = sched[k - 1]
            handles[pb].wait()
            pltpu.sync_copy(row_bufs[pb],
                            pdst.at[pl.ds(base + pc * CHUNK, CHUNK)])
    lb = (len(sched) - 1) & 1
    _, ldst, lc = sched[-1]
    handles[lb].wait()
    pltpu.sync_copy(row_bufs[lb], ldst.at[pl.ds(base + lc * CHUNK, CHUNK)])


_sc_kernel = functools.partial(
    pl.kernel,
    out_type=(jax.ShapeDtypeStruct((N, EMBED_DIM), jnp.float32),
              jax.ShapeDtypeStruct((N, EMBED_DIM), jnp.float32)),
    mesh=_MESH,
    scratch_types=[
        pltpu.VMEM((CHUNK,), jnp.int32),
        pltpu.VMEM((CHUNK,), jnp.int32),
        pltpu.VMEM((CHUNK, EMBED_DIM), jnp.float32),
        pltpu.VMEM((CHUNK, EMBED_DIM), jnp.float32),
        pltpu.SemaphoreType.DMA,
        pltpu.SemaphoreType.DMA,
    ],
    compiler_params=pltpu.CompilerParams(use_tc_tiling_on_sc=False),
)(_body)


def kernel(x1, x2, table):
    f1 = x1.reshape(-1).astype(jnp.int32)
    f2 = x2.reshape(-1).astype(jnp.int32)
    o1, o2 = _sc_kernel(f1, f2, table)
    return (o1.reshape(BATCH, HIST, EMBED_DIM),
            o2.reshape(BATCH, HIST, EMBED_DIM))
